# Initial kernel scaffold; baseline (speedup 1.0000x reference)
import jax
import jax.numpy as jnp
from jax.experimental import pallas as pl


def kernel(x, edge_index, W1, a_src1, a_dst1, b1, W2, a_src2, a_dst2, b2):
    raise NotImplementedError("write your pallas kernel here")


# speedup vs baseline: 34.6597x; 34.6597x over previous
import functools

import jax
import jax.numpy as jnp
from jax import lax
from jax.experimental import pallas as pl
from jax.experimental.pallas import tpu as pltpu
from jax.experimental.pallas import tpu_sc as plsc

N = 10000
E = 320000
IN_DIM = 128
HID = 16
HEADS = 8
D1 = HEADS * HID
OUT_DIM = 40
D2P = 48

NC = 2
NS = 16
NW = NC * NS
EPW = E // NW
K = 80
NCHUNK = EPW // K
RPT = 624
REM0 = NS * RPT
REM = N - REM0


def _leaky(v):
    return jnp.where(v >= 0, v, 0.2 * v)


def _bcast_lane(w, lane):
    idx = jnp.full((16, 1), lane, jnp.int32)
    dn = lax.GatherDimensionNumbers(
        offset_dims=(), collapsed_slice_dims=(0,), start_index_map=(0,))
    return lax.gather(w, idx, dn, (1,),
                      mode=lax.GatherScatterMode.PROMISE_IN_BOUNDS)


def _proj1_body(x_ref, w1_ref, as_ref, ad_ref, h_ref, asrc_ref, adst_ref):
    h = jnp.dot(x_ref[...], w1_ref[...], preferred_element_type=jnp.float32)
    h_ref[...] = h
    asrc_ref[...] = jnp.dot(h, as_ref[...], preferred_element_type=jnp.float32)
    adst_ref[...] = jnp.dot(h, ad_ref[...], preferred_element_type=jnp.float32)


def _proj1(x, W1, As, Ad):
    return pl.pallas_call(
        _proj1_body,
        out_shape=[
            jax.ShapeDtypeStruct((N, D1), jnp.float32),
            jax.ShapeDtypeStruct((N, HEADS), jnp.float32),
            jax.ShapeDtypeStruct((N, HEADS), jnp.float32),
        ],
    )(x, W1, As, Ad)


def _edge1_body(src_hbm, dst_hbm, asp_hbm, adp_hbm, h_hbm, z128_hbm, z16_hbm,
                numer_out, denom_out,
                src_v, dst_v, as_rows, ad_rows, h_rows, w_buf, msg_buf,
                numer_sp, denom_sp, sem1, sem2, sem3):
    c = lax.axis_index("c")
    s = lax.axis_index("s")

    pltpu.sync_copy(z128_hbm, numer_sp.at[pl.ds(s * RPT, RPT)])
    pltpu.sync_copy(z16_hbm, denom_sp.at[pl.ds(s * RPT, RPT)])

    @pl.when(s == NS - 1)
    def _():
        pltpu.sync_copy(z128_hbm.at[pl.ds(0, REM)], numer_sp.at[pl.ds(REM0, REM)])
        pltpu.sync_copy(z16_hbm.at[pl.ds(0, REM)], denom_sp.at[pl.ds(REM0, REM)])

    plsc.subcore_barrier()

    base0 = c * (E // NC) + s * EPW

    def chunk(i, carry):
        base = base0 + i * K
        pltpu.sync_copy(src_hbm.at[pl.ds(base, K)], src_v)
        pltpu.sync_copy(dst_hbm.at[pl.ds(base, K)], dst_v)
        cp1 = pltpu.async_copy(asp_hbm.at[src_v], as_rows, sem1)
        cp2 = pltpu.async_copy(adp_hbm.at[dst_v], ad_rows, sem2)
        cp3 = pltpu.async_copy(h_hbm.at[src_v], h_rows, sem3)
        cp1.wait()
        cp2.wait()
        cp3.wait()

        def edge(e, carry2):
            ev = as_rows[e, :] + ad_rows[e, :]
            w = jnp.exp(_leaky(ev))
            w_buf[e, :] = w
            for g in range(HEADS):
                wb = _bcast_lane(w, g)
                msg_buf[e, pl.ds(g * 16, 16)] = h_rows[e, pl.ds(g * 16, 16)] * wb
            return carry2

        lax.fori_loop(0, K, edge, 0)
        pltpu.sync_copy(msg_buf, numer_sp.at[dst_v], add=True)
        pltpu.sync_copy(w_buf, denom_sp.at[dst_v], add=True)
        return carry

    lax.fori_loop(0, NCHUNK, chunk, 0)
    plsc.subcore_barrier()

    r0 = s * RPT
    pltpu.sync_copy(numer_sp.at[pl.ds(r0, RPT)], numer_out.at[c, pl.ds(r0, RPT)])
    pltpu.sync_copy(denom_sp.at[pl.ds(r0, RPT)], denom_out.at[c, pl.ds(r0, RPT)])

    @pl.when(s == NS - 1)
    def _():
        pltpu.sync_copy(numer_sp.at[pl.ds(REM0, REM)],
                        numer_out.at[c, pl.ds(REM0, REM)])
        pltpu.sync_copy(denom_sp.at[pl.ds(REM0, REM)],
                        denom_out.at[c, pl.ds(REM0, REM)])


def _edge1(src, dst, asp, adp, h, z128, z16):
    mesh = plsc.VectorSubcoreMesh(core_axis_name="c", subcore_axis_name="s")
    f = functools.partial(
        pl.kernel,
        mesh=mesh,
        compiler_params=pltpu.CompilerParams(use_tc_tiling_on_sc=False),
        out_type=[
            jax.ShapeDtypeStruct((NC, N, D1), jnp.float32),
            jax.ShapeDtypeStruct((NC, N, 16), jnp.float32),
        ],
        scratch_types=[
            pltpu.VMEM((K,), jnp.int32),
            pltpu.VMEM((K,), jnp.int32),
            pltpu.VMEM((K, 16), jnp.float32),
            pltpu.VMEM((K, 16), jnp.float32),
            pltpu.VMEM((K, D1), jnp.float32),
            pltpu.VMEM((K, 16), jnp.float32),
            pltpu.VMEM((K, D1), jnp.float32),
            pltpu.VMEM_SHARED((N, D1), jnp.float32),
            pltpu.VMEM_SHARED((N, 16), jnp.float32),
            pltpu.SemaphoreType.DMA,
            pltpu.SemaphoreType.DMA,
            pltpu.SemaphoreType.DMA,
        ],
    )(_edge1_body)
    return f(src, dst, asp, adp, h, z128, z16)


def _mid_body(n0_ref, n1_ref, d0_ref, d1_ref, asrc_ref, adst_ref, h_ref,
              b1_ref, ex_ref, w2_ref, a2s_ref, a2d_ref,
              g2_ref, s2a_ref, s2b_ref):
    sc = jnp.exp(_leaky(asrc_ref[...] + adst_ref[...]))
    den8 = d0_ref[...] + d1_ref[...] + sc
    ex = ex_ref[...]
    num = (n0_ref[...] + n1_ref[...]
           + jnp.dot(sc, ex, preferred_element_type=jnp.float32) * h_ref[...])
    den = jnp.dot(den8, ex, preferred_element_type=jnp.float32)
    hmid = num / den + b1_ref[...]
    hmid = jnp.where(hmid > 0, hmid, jnp.exp(jnp.minimum(hmid, 0.0)) - 1.0)
    g2 = jnp.dot(hmid, w2_ref[...], preferred_element_type=jnp.float32)
    g2_ref[...] = g2
    s2a_ref[...] = jnp.dot(g2, a2s_ref[...], preferred_element_type=jnp.float32)
    s2b_ref[...] = jnp.dot(g2, a2d_ref[...], preferred_element_type=jnp.float32)


def _mid(n0, n1, d0, d1, asrc, adst, h, b1row, ex, W2, a2s, a2d):
    return pl.pallas_call(
        _mid_body,
        out_shape=[
            jax.ShapeDtypeStruct((N, OUT_DIM), jnp.float32),
            jax.ShapeDtypeStruct((N, 1), jnp.float32),
            jax.ShapeDtypeStruct((N, 1), jnp.float32),
        ],
    )(n0, n1, d0, d1, asrc, adst, h, b1row, ex, W2, a2s, a2d)


def _edge2_body(src_hbm, dst_hbm, asp_hbm, adp_hbm, g_hbm, z48_hbm, z16_hbm,
                numer_out, denom_out,
                src_v, dst_v, as_rows, ad_rows, g_rows, w_buf, msg_buf,
                numer_sp, denom_sp, sem1, sem2, sem3):
    c = lax.axis_index("c")
    s = lax.axis_index("s")

    pltpu.sync_copy(z48_hbm, numer_sp.at[pl.ds(s * RPT, RPT)])
    pltpu.sync_copy(z16_hbm, denom_sp.at[pl.ds(s * RPT, RPT)])

    @pl.when(s == NS - 1)
    def _():
        pltpu.sync_copy(z48_hbm.at[pl.ds(0, REM)], numer_sp.at[pl.ds(REM0, REM)])
        pltpu.sync_copy(z16_hbm.at[pl.ds(0, REM)], denom_sp.at[pl.ds(REM0, REM)])

    plsc.subcore_barrier()

    base0 = c * (E // NC) + s * EPW

    def chunk(i, carry):
        base = base0 + i * K
        pltpu.sync_copy(src_hbm.at[pl.ds(base, K)], src_v)
        pltpu.sync_copy(dst_hbm.at[pl.ds(base, K)], dst_v)
        cp1 = pltpu.async_copy(asp_hbm.at[src_v], as_rows, sem1)
        cp2 = pltpu.async_copy(adp_hbm.at[dst_v], ad_rows, sem2)
        cp3 = pltpu.async_copy(g_hbm.at[src_v], g_rows, sem3)
        cp1.wait()
        cp2.wait()
        cp3.wait()

        def edge(e, carry2):
            ev = as_rows[e, :] + ad_rows[e, :]
            w = jnp.exp(_leaky(ev))
            w_buf[e, :] = w
            wb = _bcast_lane(w, 0)
            for g in range(3):
                msg_buf[e, pl.ds(g * 16, 16)] = g_rows[e, pl.ds(g * 16, 16)] * wb
            return carry2

        lax.fori_loop(0, K, edge, 0)
        pltpu.sync_copy(msg_buf, numer_sp.at[dst_v], add=True)
        pltpu.sync_copy(w_buf, denom_sp.at[dst_v], add=True)
        return carry

    lax.fori_loop(0, NCHUNK, chunk, 0)
    plsc.subcore_barrier()

    r0 = s * RPT
    pltpu.sync_copy(numer_sp.at[pl.ds(r0, RPT)], numer_out.at[c, pl.ds(r0, RPT)])
    pltpu.sync_copy(denom_sp.at[pl.ds(r0, RPT)], denom_out.at[c, pl.ds(r0, RPT)])

    @pl.when(s == NS - 1)
    def _():
        pltpu.sync_copy(numer_sp.at[pl.ds(REM0, REM)],
                        numer_out.at[c, pl.ds(REM0, REM)])
        pltpu.sync_copy(denom_sp.at[pl.ds(REM0, REM)],
                        denom_out.at[c, pl.ds(REM0, REM)])


def _edge2(src, dst, asp, adp, g2, z48, z16):
    mesh = plsc.VectorSubcoreMesh(core_axis_name="c", subcore_axis_name="s")
    f = functools.partial(
        pl.kernel,
        mesh=mesh,
        compiler_params=pltpu.CompilerParams(use_tc_tiling_on_sc=False),
        out_type=[
            jax.ShapeDtypeStruct((NC, N, D2P), jnp.float32),
            jax.ShapeDtypeStruct((NC, N, 16), jnp.float32),
        ],
        scratch_types=[
            pltpu.VMEM((K,), jnp.int32),
            pltpu.VMEM((K,), jnp.int32),
            pltpu.VMEM((K, 16), jnp.float32),
            pltpu.VMEM((K, 16), jnp.float32),
            pltpu.VMEM((K, D2P), jnp.float32),
            pltpu.VMEM((K, 16), jnp.float32),
            pltpu.VMEM((K, D2P), jnp.float32),
            pltpu.VMEM_SHARED((N, D2P), jnp.float32),
            pltpu.VMEM_SHARED((N, 16), jnp.float32),
            pltpu.SemaphoreType.DMA,
            pltpu.SemaphoreType.DMA,
            pltpu.SemaphoreType.DMA,
        ],
    )(_edge2_body)
    return f(src, dst, asp, adp, g2, z48, z16)


def _fin_body(n0_ref, n1_ref, d0_ref, d1_ref, s2a_ref, s2b_ref, g2_ref,
              b2_ref, out_ref):
    sc = jnp.exp(_leaky(s2a_ref[...] + s2b_ref[...]))
    num = n0_ref[...] + n1_ref[...] + sc * g2_ref[...]
    den = d0_ref[...] + d1_ref[...] + sc
    out_ref[...] = num / den + b2_ref[...]


def _fin(n0, n1, d0, d1, s2a, s2b, g2, b2row):
    return pl.pallas_call(
        _fin_body,
        out_shape=jax.ShapeDtypeStruct((N, OUT_DIM), jnp.float32),
    )(n0, n1, d0, d1, s2a, s2b, g2, b2row)


def kernel(x, edge_index, W1, a_src1, a_dst1, b1, W2, a_src2, a_dst2, b2):
    src = edge_index[0]
    dst = edge_index[1]

    eye8 = jnp.eye(HEADS, dtype=jnp.float32)
    As = (eye8[:, None, :] * a_src1[:, :, None]).reshape(D1, HEADS)
    Ad = (eye8[:, None, :] * a_dst1[:, :, None]).reshape(D1, HEADS)

    h1, asrc1, adst1 = _proj1(x, W1, As, Ad)

    pad8 = jnp.zeros((N, 8), jnp.float32)
    asp1 = jnp.concatenate([asrc1, pad8], axis=1)
    adp1 = jnp.concatenate([adst1, pad8], axis=1)

    z128 = jnp.zeros((RPT, D1), jnp.float32)
    z16 = jnp.zeros((RPT, 16), jnp.float32)
    z48 = jnp.zeros((RPT, D2P), jnp.float32)

    numer1, denom1 = _edge1(src, dst, asp1, adp1, h1, z128, z16)

    ex = jnp.repeat(eye8, HID, axis=1)
    g2, s2a, s2b = _mid(
        numer1[0], numer1[1],
        denom1[0, :, :HEADS], denom1[1, :, :HEADS],
        asrc1, adst1, h1, b1[None, :], ex, W2,
        a_src2[0][:, None], a_dst2[0][:, None])

    pad15 = jnp.zeros((N, 15), jnp.float32)
    asp2 = jnp.concatenate([s2a, pad15], axis=1)
    adp2 = jnp.concatenate([s2b, pad15], axis=1)
    g2p = jnp.concatenate([g2, jnp.zeros((N, D2P - OUT_DIM), jnp.float32)], axis=1)

    numer2, denom2 = _edge2(src, dst, asp2, adp2, g2p, z48, z16)

    out = _fin(numer2[0, :, :OUT_DIM], numer2[1, :, :OUT_DIM],
               denom2[0, :, :1], denom2[1, :, :1],
               s2a, s2b, g2, b2[None, :])
    return out
